# baseline (speedup 1.0000x reference)
import functools

import jax
import jax.numpy as jnp
from jax import lax
from jax.experimental import pallas as pl
from jax.experimental.pallas import tpu as pltpu
from jax.experimental.pallas import tpu_sc as plsc

B, L, D = 4096, 200, 64
DAY_SIZE, TIME_SIZE = 32, 32
OUT_D = D + DAY_SIZE + TIME_SIZE

N = B * L
NC, NS = 2, 16
NW = NC * NS
TPW = N // NW
TILE = 512
NTILES = TPW // TILE
GCH = 128
NG = TILE // GCH


def _sc_body(inp_hbm, didx_hbm, tidx_hbm, day_hbm, time_hbm, out_hbm,
             didx_v, tidx_v, day_v, time_v, gsem, osem):
    wid = lax.axis_index("s") * NC + lax.axis_index("c")
    row0 = wid * (TPW // GCH)

    def tile_step(t, _):
        base = wid * TPW + t * TILE
        irow = row0 + t * NG
        pltpu.sync_copy(didx_hbm.at[pl.ds(irow, NG)], didx_v)
        pltpu.sync_copy(tidx_hbm.at[pl.ds(irow, NG)], tidx_v)
        for j in range(NG):
            pltpu.async_copy(day_hbm.at[didx_v.at[j]],
                             day_v.at[pl.ds(j * GCH, GCH)], gsem)
            pltpu.async_copy(time_hbm.at[tidx_v.at[j]],
                             time_v.at[pl.ds(j * GCH, GCH)], gsem)
        pltpu.async_copy(inp_hbm.at[pl.ds(base, TILE)],
                         out_hbm.at[pl.ds(base, TILE), pl.ds(0, D)], osem)
        for j in range(NG):
            pltpu.make_async_copy(day_hbm.at[didx_v.at[j]],
                                  day_v.at[pl.ds(j * GCH, GCH)], gsem).wait()
            pltpu.make_async_copy(time_hbm.at[tidx_v.at[j]],
                                  time_v.at[pl.ds(j * GCH, GCH)], gsem).wait()
        pltpu.sync_copy(day_v, out_hbm.at[pl.ds(base, TILE), pl.ds(D, DAY_SIZE)])
        pltpu.sync_copy(time_v,
                        out_hbm.at[pl.ds(base, TILE), pl.ds(D + DAY_SIZE, TIME_SIZE)])
        pltpu.make_async_copy(inp_hbm.at[pl.ds(base, TILE)],
                              out_hbm.at[pl.ds(base, TILE), pl.ds(0, D)],
                              osem).wait()
        return ()

    lax.fori_loop(0, NTILES, tile_step, (), unroll=False)


@functools.partial(jax.jit, donate_argnums=())
def _run(inp2, didx, tidx, emb_day, emb_time):
    kern = pl.kernel(
        _sc_body,
        out_type=jax.ShapeDtypeStruct((N, OUT_D), jnp.float32),
        mesh=plsc.VectorSubcoreMesh(core_axis_name="c", subcore_axis_name="s"),
        scratch_types=[
            pltpu.VMEM((NG, GCH), jnp.int32),
            pltpu.VMEM((NG, GCH), jnp.int32),
            pltpu.VMEM((TILE, DAY_SIZE), jnp.float32),
            pltpu.VMEM((TILE, TIME_SIZE), jnp.float32),
            pltpu.SemaphoreType.DMA,
            pltpu.SemaphoreType.DMA,
        ],
        compiler_params=pltpu.CompilerParams(use_tc_tiling_on_sc=False),
    )
    return kern(inp2, didx, tidx, emb_day, emb_time)


def kernel(inp, daytime, emb_day, emb_time):
    inp2 = inp.reshape(N, D)
    dt = daytime.astype(jnp.int32)
    didx = dt[:, :, 0].reshape(N // GCH, GCH)
    tidx = dt[:, :, 1].reshape(N // GCH, GCH)
    out = _run(inp2, didx, tidx, emb_day, emb_time)
    return out.reshape(B, L, OUT_D)
